# baseline (speedup 1.0000x reference)
import jax
import jax.numpy as jnp
from jax import lax
from jax.experimental import pallas as pl
from jax.experimental.pallas import tpu as pltpu
from jax.experimental.pallas import tpu_sc as plsc

_B = 4096
_D = 64
_L = 16
_NG = _B // _L
_SMALL_V = 1000
_BIG_V = 100000
_CH = 9088
_NFULL = 11
_TAIL0 = _NFULL * _CH


def _call1_body(ints_hbm, *rest):
    tabs = rest[:24]
    tails = rest[24:30]
    out_hbm = rest[30]
    idx_v, col_s, col_b, tail_v, acc_v = rest[31:]

    cid = lax.axis_index("c")
    sid = lax.axis_index("s")
    wid = sid * 2 + cid

    def idx16(gi):
        return idx_v[lax.div(gi, 8), pl.ds(lax.rem(gi, 8) * _L, _L)]

    for s in range(18):
        g = lax.rem(wid - 8 * s + 256, 32)

        @pl.when(g < 8)
        def _(s=s, g=g):
            pltpu.sync_copy(ints_hbm.at[s], idx_v)
            pltpu.sync_copy(tabs[s].at[pl.ds(g * 8, 8)], col_s)

            def grp(gi, _):
                sl = pl.ds(gi * _L, _L)
                v = jnp.minimum(jnp.maximum(idx16(gi), 0), _SMALL_V)
                for i in range(8):
                    sub = jnp.full((_L,), i, jnp.int32)
                    acc_v[i, sl] = plsc.load_gather(col_s, [sub, v])
                return 0

            lax.fori_loop(0, _NG, grp, 0)
            pltpu.sync_copy(acc_v, out_hbm.at[pl.ds(s * _D + g * 8, 8)])

    for si in range(6):
        s = 18 + si
        g = lax.rem(wid - 8 * si + 256, 32)

        @pl.when(g < 8)
        def _(s=s, si=si, g=g):
            pltpu.sync_copy(ints_hbm.at[s], idx_v)
            pltpu.sync_copy(tails[si], tail_v)

            def zero(gi, _):
                sl = pl.ds(gi * _L, _L)
                zz = jnp.zeros((_L,), jnp.float32)
                for i in range(8):
                    acc_v[i, sl] = zz
                return 0

            lax.fori_loop(0, _NG, zero, 0)

            def chunk(c, _, s=s, g=g):
                c0 = c * _CH
                pltpu.sync_copy(
                    tabs[s].at[pl.ds(g * 8, 8), pl.ds(c0, _CH)], col_b
                )

                def grp(gi, _, c0=c0):
                    sl = pl.ds(gi * _L, _L)
                    v = idx16(gi)
                    m = (v >= c0) & (v < c0 + _CH)
                    r = jnp.minimum(jnp.maximum(v - c0, 0), _CH - 1)
                    zz = jnp.zeros((_L,), jnp.float32)
                    for i in range(8):
                        sub = jnp.full((_L,), i, jnp.int32)
                        gth = plsc.load_gather(col_b, [sub, r])
                        plsc.addupdate(acc_v.at[i, sl], jnp.where(m, gth, zz))
                    return 0

                lax.fori_loop(0, _NG, grp, 0)
                return 0

            lax.fori_loop(0, _NFULL, chunk, 0)

            def tgrp(gi, _, g=g):
                sl = pl.ds(gi * _L, _L)
                v = jnp.minimum(jnp.maximum(idx16(gi), 0), _BIG_V)
                m = v >= _TAIL0
                lr = jnp.minimum(jnp.maximum(v - _TAIL0, 0), 39)
                zz = jnp.zeros((_L,), jnp.float32)
                for i in range(8):
                    dcol = jnp.full((_L,), 0, jnp.int32) + (g * 8 + i)
                    gth = plsc.load_gather(tail_v, [lr, dcol])
                    plsc.addupdate(acc_v.at[i, sl], jnp.where(m, gth, zz))
                return 0

            lax.fori_loop(0, _NG, tgrp, 0)
            pltpu.sync_copy(acc_v, out_hbm.at[pl.ds(s * _D + g * 8, 8)])


def _call2_body(ints_hbm, t24, t25, out_hbm, ints_v, bag_v, acc_v, outt_v, sem):
    cid = lax.axis_index("c")
    sid = lax.axis_index("s")
    wid = sid * 2 + cid
    base = wid * 128

    pltpu.sync_copy(ints_hbm.at[pl.ds(24, 40), pl.ds(base, 128)], ints_v)

    lanes = lax.iota(jnp.int32, _L)
    for sn in range(2):
        for j in range(20):
            pltpu.async_copy(
                (t24 if sn == 0 else t25).at[ints_v.at[sn * 20 + j]],
                bag_v,
                sem,
            ).wait()

            def addg(r, _, j=j):
                for dd in range(4):
                    sl = pl.ds(dd * _L, _L)
                    x = bag_v[r, sl]
                    if j == 0:
                        acc_v[r, sl] = x
                    else:
                        plsc.addupdate(acc_v.at[r, sl], x)
                return 0

            lax.fori_loop(0, 128, addg, 0)

        inv = jnp.float32(1.0 / 20.0)

        def tr(r, _):
            col = jnp.full((_L,), 0, jnp.int32) + r
            for dd in range(4):
                sl = pl.ds(dd * _L, _L)
                x = acc_v[r, sl] * inv
                plsc.store_scatter(outt_v, [dd * _L + lanes, col], x)
            return 0

        lax.fori_loop(0, 128, tr, 0)
        pltpu.sync_copy(
            outt_v, out_hbm.at[pl.ds(sn * _D, _D), pl.ds(base, 128)]
        )


def kernel(int_feats, tables):
    ints_t = jnp.transpose(int_feats)
    smalls = tuple(
        jnp.transpose(jnp.pad(t, ((0, 23), (0, 0)))) for t in tables[:18]
    )
    bigs_t = tuple(jnp.transpose(t) for t in tables[18:24])
    tails = tuple(
        jnp.pad(t[_TAIL0:], ((0, 7), (0, 64))) for t in tables[18:24]
    )
    bag_tabs = [jnp.pad(t, ((0, 7), (0, 64))) for t in tables[24:]]
    ints_p = jnp.reshape(ints_t, (64, 32, 128))

    cp = pltpu.CompilerParams(use_tc_tiling_on_sc=True, needs_layout_passes=False)
    mesh = plsc.VectorSubcoreMesh(core_axis_name="c", subcore_axis_name="s")

    call1 = pl.kernel(
        _call1_body,
        out_type=jax.ShapeDtypeStruct((24 * _D, _B), jnp.float32),
        mesh=mesh,
        compiler_params=cp,
        scratch_types=[
            pltpu.VMEM((32, 128), jnp.int32),
            pltpu.VMEM((8, 1024), jnp.float32),
            pltpu.VMEM((8, _CH), jnp.float32),
            pltpu.VMEM((40, 128), jnp.float32),
            pltpu.VMEM((8, _B), jnp.float32),
        ],
    )
    call2 = pl.kernel(
        _call2_body,
        out_type=jax.ShapeDtypeStruct((2 * _D, _B), jnp.float32),
        mesh=mesh,
        compiler_params=cp,
        scratch_types=[
            pltpu.VMEM((40, 128), jnp.int32),
            pltpu.VMEM((128, 128), jnp.float32),
            pltpu.VMEM((128, _D), jnp.float32),
            pltpu.VMEM((_D, 128), jnp.float32),
            pltpu.SemaphoreType.DMA,
        ],
    )
    o1 = call1(ints_p, *smalls, *bigs_t, *tails)
    o2 = call2(ints_t, bag_tabs[0], bag_tabs[1])
    out = jnp.concatenate([o1, o2], axis=0)
    return jnp.transpose(out.reshape(26, _D, _B), (2, 0, 1))


# speedup vs baseline: 1.2136x; 1.2136x over previous
import jax
import jax.numpy as jnp
from jax import lax
from jax.experimental import pallas as pl
from jax.experimental.pallas import tpu as pltpu
from jax.experimental.pallas import tpu_sc as plsc

_B = 4096
_D = 64
_L = 16
_NG = _B // _L
_SMALL_V = 1000


def _call1_body(ints_hbm, *rest):
    tabs = rest[:18]
    out_hbm = rest[18]
    idx_v, col_s, acc_v = rest[19:]

    cid = lax.axis_index("c")
    sid = lax.axis_index("s")
    wid = sid * 2 + cid

    def idx16(gi):
        return idx_v[lax.div(gi, 8), pl.ds(lax.rem(gi, 8) * _L, _L)]

    for s in range(18):
        g = lax.rem(wid - 8 * s + 256, 32)

        @pl.when(g < 8)
        def _(s=s, g=g):
            pltpu.sync_copy(ints_hbm.at[s], idx_v)
            pltpu.sync_copy(tabs[s].at[pl.ds(g * 8, 8)], col_s)

            def grp(gi, _):
                sl = pl.ds(gi * _L, _L)
                v = jnp.minimum(jnp.maximum(idx16(gi), 0), _SMALL_V)
                for i in range(8):
                    sub = jnp.full((_L,), i, jnp.int32)
                    acc_v[i, sl] = plsc.load_gather(col_s, [sub, v])
                return 0

            lax.fori_loop(0, _NG, grp, 0)
            pltpu.sync_copy(acc_v, out_hbm.at[pl.ds(s * _D + g * 8, 8)])


def _call2_body(ints_hbm, *rest):
    tabs = rest[:8]
    out_hbm = rest[8]
    ints_v = rest[9]
    bufs = rest[10:12]
    acc_v = rest[12]
    outt_v = rest[13]
    sems = rest[14:16]

    cid = lax.axis_index("c")
    sid = lax.axis_index("s")
    wid = sid * 2 + cid
    base = wid * 128

    pltpu.sync_copy(ints_hbm.at[pl.ds(16, 48), pl.ds(base, 128)], ints_v)

    lanes = lax.iota(jnp.int32, _L)

    rounds = [(si, si, 2 + si) for si in range(6)]
    for sn in range(2):
        rounds += [(6 + sn, 6 + sn, 8 + sn * 20 + j) for j in range(20)]

    def issue(k):
        t_i, _, row = rounds[k]
        return pltpu.async_copy(
            tabs[t_i].at[ints_v.at[row]], bufs[k % 2], sems[k % 2]
        )

    def transpose_out(src_ref, scale, out_blk):
        def tr(r, _):
            col = jnp.full((_L,), 0, jnp.int32) + r
            for dd in range(4):
                x = src_ref[r, pl.ds(dd * _L, _L)]
                if scale is not None:
                    x = x * scale
                plsc.store_scatter(outt_v, [dd * _L + lanes, col], x)
            return 0

        lax.fori_loop(0, 128, tr, 0)
        pltpu.sync_copy(
            outt_v, out_hbm.at[pl.ds(out_blk * _D, _D), pl.ds(base, 128)]
        )

    cp = issue(0)
    inv = jnp.float32(1.0 / 20.0)
    for k in range(46):
        nxt = issue(k + 1) if k + 1 < 46 else None
        cp.wait()
        buf = bufs[k % 2]
        if k < 6:
            transpose_out(buf, None, rounds[k][1])
        else:
            j = (k - 6) % 20

            def addg(r, _, j=j, buf=buf):
                for dd in range(4):
                    sl = pl.ds(dd * _L, _L)
                    x = buf[r, sl]
                    if j == 0:
                        acc_v[r, sl] = x
                    else:
                        plsc.addupdate(acc_v.at[r, sl], x)
                return 0

            lax.fori_loop(0, 128, addg, 0)
            if j == 19:
                transpose_out(acc_v, inv, rounds[k][1])
        cp = nxt


def kernel(int_feats, tables):
    ints_t = jnp.transpose(int_feats)
    smalls = tuple(
        jnp.transpose(jnp.pad(t, ((0, 23), (0, 0)))) for t in tables[:18]
    )
    bigs = [jnp.pad(t, ((0, 7), (0, 64))) for t in tables[18:]]
    ints_p = jnp.reshape(ints_t, (64, 32, 128))

    cp = pltpu.CompilerParams(use_tc_tiling_on_sc=True, needs_layout_passes=False)
    mesh = plsc.VectorSubcoreMesh(core_axis_name="c", subcore_axis_name="s")

    call1 = pl.kernel(
        _call1_body,
        out_type=jax.ShapeDtypeStruct((18 * _D, _B), jnp.float32),
        mesh=mesh,
        compiler_params=cp,
        scratch_types=[
            pltpu.VMEM((32, 128), jnp.int32),
            pltpu.VMEM((8, 1024), jnp.float32),
            pltpu.VMEM((8, _B), jnp.float32),
        ],
    )
    call2 = pl.kernel(
        _call2_body,
        out_type=jax.ShapeDtypeStruct((8 * _D, _B), jnp.float32),
        mesh=mesh,
        compiler_params=cp,
        scratch_types=[
            pltpu.VMEM((48, 128), jnp.int32),
            pltpu.VMEM((128, 128), jnp.float32),
            pltpu.VMEM((128, 128), jnp.float32),
            pltpu.VMEM((128, _D), jnp.float32),
            pltpu.VMEM((_D, 128), jnp.float32),
            pltpu.SemaphoreType.DMA,
            pltpu.SemaphoreType.DMA,
        ],
    )
    o1 = call1(ints_p, *smalls)
    o2 = call2(ints_t, *bigs)
    out = jnp.concatenate([o1, o2], axis=0)
    return jnp.transpose(out.reshape(26, _D, _B), (2, 0, 1))
